# speedup vs baseline: 1.8963x; 1.8963x over previous
import jax
import jax.numpy as jnp
from jax.experimental import pallas as pl
from jax.experimental.pallas import tpu as pltpu

BLOCK_ROWS = 512


def _softmax_block(x_ref, o_ref):
    x = x_ref[...]
    m = jnp.max(x, axis=-1, keepdims=True)
    e = jnp.exp(x - m)
    s = jnp.sum(e, axis=-1, keepdims=True)
    o_ref[...] = e * (1.0 / s)


def kernel(attn_weights, input_ids):
    del input_ids
    shape = attn_weights.shape
    n_rows = shape[0] * shape[1] * shape[2]
    x = attn_weights.reshape(n_rows, shape[3])

    out = pl.pallas_call(
        _softmax_block,
        grid=(n_rows // BLOCK_ROWS,),
        in_specs=[pl.BlockSpec((BLOCK_ROWS, shape[3]), lambda i: (i, 0))],
        out_specs=pl.BlockSpec((BLOCK_ROWS, shape[3]), lambda i: (i, 0)),
        out_shape=jax.ShapeDtypeStruct((n_rows, shape[3]), x.dtype),
        compiler_params=pltpu.CompilerParams(
            dimension_semantics=("parallel",),
        ),
    )(x)
    return out.reshape(shape)


# speedup vs baseline: 1.9323x; 1.0190x over previous
import jax
import jax.numpy as jnp
from jax.experimental import pallas as pl
from jax.experimental.pallas import tpu as pltpu

BLOCK_ROWS = 1024


def _softmax_block(x_ref, o_ref):
    x = x_ref[...]
    m = jnp.max(x, axis=-1, keepdims=True)
    e = jnp.exp(x - m)
    s = jnp.sum(e, axis=-1, keepdims=True)
    o_ref[...] = e * (1.0 / s)


def kernel(attn_weights, input_ids):
    del input_ids
    shape = attn_weights.shape
    n_rows = shape[0] * shape[1] * shape[2]
    x = attn_weights.reshape(n_rows, shape[3])

    out = pl.pallas_call(
        _softmax_block,
        grid=(n_rows // BLOCK_ROWS,),
        in_specs=[pl.BlockSpec((BLOCK_ROWS, shape[3]), lambda i: (i, 0))],
        out_specs=pl.BlockSpec((BLOCK_ROWS, shape[3]), lambda i: (i, 0)),
        out_shape=jax.ShapeDtypeStruct((n_rows, shape[3]), x.dtype),
        compiler_params=pltpu.CompilerParams(
            dimension_semantics=("parallel",),
        ),
    )(x)
    return out.reshape(shape)
